# baseline (speedup 1.0000x reference)
import functools

import jax
import jax.numpy as jnp
from jax import lax
from jax.experimental import pallas as pl
from jax.experimental.pallas import tpu as pltpu
from jax.experimental.pallas import tpu_sc as plsc


def _sc_gather(packed, idx, n_rows, row_w):
    NW = 32
    rows_per_w = n_rows // NW
    STREAM = 128
    K = 5
    GROUP = K * STREAM
    NGRP = rows_per_w // GROUP

    mesh = plsc.VectorSubcoreMesh(core_axis_name="c", subcore_axis_name="s")

    @functools.partial(
        pl.kernel,
        mesh=mesh,
        out_type=jax.ShapeDtypeStruct((n_rows, row_w), jnp.float32),
        scratch_types=[
            pltpu.VMEM((rows_per_w,), jnp.int32),
            pltpu.VMEM((GROUP, row_w), jnp.float32),
            pltpu.SemaphoreType.DMA,
        ],
        compiler_params=pltpu.CompilerParams(use_tc_tiling_on_sc=False),
    )
    def gather_k(tab_hbm, idx_hbm, out_hbm, idx_v, rows_v, sem):
        cid = lax.axis_index("c")
        sid = lax.axis_index("s")
        wid = sid * 2 + cid
        base = wid * rows_per_w
        pltpu.sync_copy(idx_hbm.at[pl.ds(base, rows_per_w)], idx_v)

        def grp(g, carry):
            goff = g * GROUP
            handles = []
            for j in range(K):
                handles.append(
                    pltpu.async_copy(
                        tab_hbm.at[idx_v.at[pl.ds(goff + j * STREAM, STREAM)]],
                        rows_v.at[pl.ds(j * STREAM, STREAM)],
                        sem,
                    )
                )
            for h in handles:
                h.wait()
            pltpu.sync_copy(rows_v, out_hbm.at[pl.ds(base + goff, GROUP)])
            return carry

        lax.fori_loop(0, NGRP, grp, 0)

    return gather_k(packed, idx)


def _gru_step_body(H, S, e_ref, par_ref, w_ref, b_ref, eye_ref,
                   out_ref, hn_ref, h_scr):
    s = pl.program_id(1)

    @pl.when(s == 0)
    def _init():
        h_scr[...] = jnp.zeros_like(h_scr)

    h = h_scr[...]
    x_pair = e_ref[0]
    par = par_ref[0]
    x_t = jnp.where(par > 0.5, x_pair[:, H:2 * H], x_pair[:, 0:H])
    x_T = jax.lax.dot_general(
        eye_ref[...], x_t, (((1,), (1,)), ((), ())),
        preferred_element_type=jnp.float32,
    )
    a = jnp.concatenate([x_T, h], axis=0)
    g = jnp.dot(w_ref[...], a, preferred_element_type=jnp.float32) + b_ref[...]
    r = jax.nn.sigmoid(g[0:H])
    z = jax.nn.sigmoid(g[H:2 * H])
    n = jnp.tanh(g[2 * H:3 * H] + r * g[3 * H:4 * H])
    h_new = (1.0 - z) * n + z * h
    h_scr[...] = h_new
    out_ref[0] = h_new

    @pl.when(s == S - 1)
    def _fin():
        hn_ref[0] = h_new


def _gru_tc(e_sbp, par, w2, b2, eye, B, S, H, BB):
    NB = B // BB
    body = functools.partial(_gru_step_body, H, S)
    return pl.pallas_call(
        body,
        grid=(NB, S),
        in_specs=[
            pl.BlockSpec((1, BB, 2 * H), lambda b, s: (s, b, 0)),
            pl.BlockSpec((1, BB, 1), lambda b, s: (s, b, 0)),
            pl.BlockSpec((4 * H, 2 * H), lambda b, s: (0, 0)),
            pl.BlockSpec((4 * H, 1), lambda b, s: (0, 0)),
            pl.BlockSpec((H, H), lambda b, s: (0, 0)),
        ],
        out_specs=[
            pl.BlockSpec((1, H, BB), lambda b, s: (s, 0, b)),
            pl.BlockSpec((1, H, BB), lambda b, s: (0, 0, b)),
        ],
        out_shape=[
            jax.ShapeDtypeStruct((S, H, B), jnp.float32),
            jax.ShapeDtypeStruct((1, H, B), jnp.float32),
        ],
        scratch_shapes=[pltpu.VMEM((H, BB), jnp.float32)],
        compiler_params=pltpu.CompilerParams(
            dimension_semantics=("parallel", "arbitrary"),
        ),
    )(e_sbp, par, w2, b2, eye)


def kernel(x, emb, W_ih, W_hh, b_ih, b_hh):
    B, S = x.shape
    V, H = emb.shape

    packed = emb.reshape(V // 2, 2 * H)

    xt = x.T
    idx = lax.shift_right_logical(xt, 1).reshape(-1).astype(jnp.int32)
    par = jnp.bitwise_and(xt, 1).astype(jnp.float32).reshape(S, B, 1)

    e_flat = _sc_gather(packed, idx, B * S, 2 * H)
    e_sbp = e_flat.reshape(S, B, 2 * H)

    zeros = jnp.zeros((H, H), jnp.float32)
    left = jnp.concatenate([W_ih[:2 * H], W_ih[2 * H:], zeros], axis=0)
    right = jnp.concatenate([W_hh[:2 * H], zeros, W_hh[2 * H:]], axis=0)
    w2 = jnp.concatenate([left, right], axis=1)
    b2 = jnp.concatenate(
        [b_ih[:2 * H] + b_hh[:2 * H], b_ih[2 * H:], b_hh[2 * H:]]
    )[:, None]
    eye = jnp.eye(H, dtype=jnp.float32)

    BB = min(512, B)
    out_shb, hn_hb = _gru_tc(e_sbp, par, w2, b2, eye, B, S, H, BB)
    out = jnp.transpose(out_shb, (2, 0, 1))
    h_n = jnp.transpose(hn_hb, (0, 2, 1))
    return out, h_n


# speedup vs baseline: 1.4911x; 1.4911x over previous
import functools

import jax
import jax.numpy as jnp
from jax import lax
from jax.experimental import pallas as pl
from jax.experimental.pallas import tpu as pltpu
from jax.experimental.pallas import tpu_sc as plsc


def _sc_gather(packed, idx, n_rows, row_w):
    NW = 32
    rows_per_w = n_rows // NW
    STREAM = 128
    K = 5
    GROUP = K * STREAM
    NGRP = rows_per_w // GROUP

    mesh = plsc.VectorSubcoreMesh(core_axis_name="c", subcore_axis_name="s")

    @functools.partial(
        pl.kernel,
        mesh=mesh,
        out_type=jax.ShapeDtypeStruct((n_rows, row_w), jnp.float32),
        scratch_types=[
            pltpu.VMEM((rows_per_w,), jnp.int32),
            pltpu.VMEM((GROUP, row_w), jnp.float32),
            pltpu.SemaphoreType.DMA,
        ],
        compiler_params=pltpu.CompilerParams(use_tc_tiling_on_sc=False),
    )
    def gather_k(tab_hbm, idx_hbm, out_hbm, idx_v, rows_v, sem):
        cid = lax.axis_index("c")
        sid = lax.axis_index("s")
        wid = sid * 2 + cid
        base = wid * rows_per_w
        pltpu.sync_copy(idx_hbm.at[pl.ds(base, rows_per_w)], idx_v)

        def grp(g, carry):
            goff = g * GROUP
            handles = []
            for j in range(K):
                handles.append(
                    pltpu.async_copy(
                        tab_hbm.at[idx_v.at[pl.ds(goff + j * STREAM, STREAM)]],
                        rows_v.at[pl.ds(j * STREAM, STREAM)],
                        sem,
                    )
                )
            for h in handles:
                h.wait()
            pltpu.sync_copy(rows_v, out_hbm.at[pl.ds(base + goff, GROUP)])
            return carry

        lax.fori_loop(0, NGRP, grp, 0)

    return gather_k(packed, idx)


def _pack_body(cw, h, et_ref, eye_ref, o_ref, t_scr):
    chunk = et_ref[...]
    t_scr[...] = jax.lax.dot_general(
        chunk, eye_ref[...], (((0,), (0,)), ((), ())),
        preferred_element_type=jnp.float32,
    )
    even = t_scr[pl.Slice(0, cw // 2, 2), :]
    odd = t_scr[pl.Slice(1, cw // 2, 2), :]
    o_ref[...] = jnp.concatenate([even, odd], axis=1)


def _tc_pack(embT, eye, V, H):
    CW = 12800
    grid = (V + CW - 1) // CW
    return pl.pallas_call(
        functools.partial(_pack_body, CW, H),
        grid=(grid,),
        in_specs=[
            pl.BlockSpec((H, CW), lambda i: (0, i)),
            pl.BlockSpec((H, H), lambda i: (0, 0)),
        ],
        out_specs=pl.BlockSpec((CW // 2, 2 * H), lambda i: (i, 0)),
        out_shape=jax.ShapeDtypeStruct((V // 2, 2 * H), jnp.float32),
        scratch_shapes=[pltpu.VMEM((CW, H), jnp.float32)],
        compiler_params=pltpu.CompilerParams(
            dimension_semantics=("arbitrary",),
        ),
    )(embT, eye)


def _gru_step_body(H, S, e_ref, par_ref, w_ref, b_ref, eye_ref,
                   out_ref, hn_ref, h_scr):
    s = pl.program_id(1)

    @pl.when(s == 0)
    def _init():
        h_scr[...] = jnp.zeros_like(h_scr)

    h = h_scr[...]
    x_pair = e_ref[0]
    par = par_ref[0]
    x_t = jnp.where(par > 0.5, x_pair[:, H:2 * H], x_pair[:, 0:H])
    x_T = jax.lax.dot_general(
        eye_ref[...], x_t, (((1,), (1,)), ((), ())),
        preferred_element_type=jnp.float32,
    )
    a = jnp.concatenate([x_T, h], axis=0)
    g = jnp.dot(w_ref[...], a, preferred_element_type=jnp.float32) + b_ref[...]
    r = jax.nn.sigmoid(g[0:H])
    z = jax.nn.sigmoid(g[H:2 * H])
    n = jnp.tanh(g[2 * H:3 * H] + r * g[3 * H:4 * H])
    h_new = (1.0 - z) * n + z * h
    h_scr[...] = h_new
    out_ref[0] = h_new

    @pl.when(s == S - 1)
    def _fin():
        hn_ref[0] = h_new


def _gru_tc(e_sbp, par, w2, b2, eye, B, S, H, BB):
    NB = B // BB
    body = functools.partial(_gru_step_body, H, S)
    return pl.pallas_call(
        body,
        grid=(NB, S),
        in_specs=[
            pl.BlockSpec((1, BB, 2 * H), lambda b, s: (s, b, 0)),
            pl.BlockSpec((1, BB, 1), lambda b, s: (s, b, 0)),
            pl.BlockSpec((4 * H, 2 * H), lambda b, s: (0, 0)),
            pl.BlockSpec((4 * H, 1), lambda b, s: (0, 0)),
            pl.BlockSpec((H, H), lambda b, s: (0, 0)),
        ],
        out_specs=[
            pl.BlockSpec((1, H, BB), lambda b, s: (s, 0, b)),
            pl.BlockSpec((1, H, BB), lambda b, s: (0, 0, b)),
        ],
        out_shape=[
            jax.ShapeDtypeStruct((S, H, B), jnp.float32),
            jax.ShapeDtypeStruct((1, H, B), jnp.float32),
        ],
        scratch_shapes=[pltpu.VMEM((H, BB), jnp.float32)],
        compiler_params=pltpu.CompilerParams(
            dimension_semantics=("parallel", "arbitrary"),
        ),
    )(e_sbp, par, w2, b2, eye)


def kernel(x, emb, W_ih, W_hh, b_ih, b_hh):
    B, S = x.shape
    V, H = emb.shape

    eye = jnp.eye(H, dtype=jnp.float32)
    packed = _tc_pack(emb.T, eye, V, H)

    xt = x.T
    idx = lax.shift_right_logical(xt, 1).reshape(-1).astype(jnp.int32)
    par = jnp.bitwise_and(xt, 1).astype(jnp.float32).reshape(S, B, 1)

    e_flat = _sc_gather(packed, idx, B * S, 2 * H)
    e_sbp = e_flat.reshape(S, B, 2 * H)

    zeros = jnp.zeros((H, H), jnp.float32)
    left = jnp.concatenate([W_ih[:2 * H], W_ih[2 * H:], zeros], axis=0)
    right = jnp.concatenate([W_hh[:2 * H], zeros, W_hh[2 * H:]], axis=0)
    w2 = jnp.concatenate([left, right], axis=1)
    b2 = jnp.concatenate(
        [b_ih[:2 * H] + b_hh[:2 * H], b_ih[2 * H:], b_hh[2 * H:]]
    )[:, None]

    BB = min(512, B)
    out_shb, hn_hb = _gru_tc(e_sbp, par, w2, b2, eye, B, S, H, BB)
    out = jnp.transpose(out_shb, (2, 0, 1))
    h_n = jnp.transpose(hn_hb, (0, 2, 1))
    return out, h_n


# speedup vs baseline: 1.7199x; 1.1535x over previous
import functools

import jax
import jax.numpy as jnp
from jax import lax
from jax.experimental import pallas as pl
from jax.experimental.pallas import tpu as pltpu
from jax.experimental.pallas import tpu_sc as plsc


def _sc_gather(packed, idx, n_rows, row_w):
    NW = 32
    rows_per_w = n_rows // NW
    STREAM = 128
    K = 5
    GROUP = K * STREAM
    NGRP = rows_per_w // GROUP

    mesh = plsc.VectorSubcoreMesh(core_axis_name="c", subcore_axis_name="s")

    @functools.partial(
        pl.kernel,
        mesh=mesh,
        out_type=jax.ShapeDtypeStruct((n_rows, row_w), jnp.float32),
        scratch_types=[
            pltpu.VMEM((rows_per_w,), jnp.int32),
            pltpu.VMEM((GROUP, row_w), jnp.float32),
            pltpu.SemaphoreType.DMA,
        ],
        compiler_params=pltpu.CompilerParams(use_tc_tiling_on_sc=False),
    )
    def gather_k(tab_hbm, idx_hbm, out_hbm, idx_v, rows_v, sem):
        cid = lax.axis_index("c")
        sid = lax.axis_index("s")
        wid = sid * 2 + cid
        base = wid * rows_per_w
        pltpu.sync_copy(idx_hbm.at[pl.ds(base, rows_per_w)], idx_v)

        def grp(g, carry):
            goff = g * GROUP
            handles = []
            for j in range(K):
                handles.append(
                    pltpu.async_copy(
                        tab_hbm.at[idx_v.at[pl.ds(goff + j * STREAM, STREAM)]],
                        rows_v.at[pl.ds(j * STREAM, STREAM)],
                        sem,
                    )
                )
            for h in handles:
                h.wait()
            pltpu.sync_copy(rows_v, out_hbm.at[pl.ds(base + goff, GROUP)])
            return carry

        lax.fori_loop(0, NGRP, grp, 0)

    return gather_k(packed, idx)


def _pack_body(cw, h, et_ref, eye_ref, o_ref, t_scr):
    chunk = et_ref[...]
    t_scr[...] = jax.lax.dot_general(
        chunk, eye_ref[...], (((0,), (0,)), ((), ())),
        preferred_element_type=jnp.float32,
    )
    even = t_scr[pl.Slice(0, cw // 2, 2), :]
    odd = t_scr[pl.Slice(1, cw // 2, 2), :]
    o_ref[...] = jnp.concatenate([even, odd], axis=1)


def _tc_pack(embT, eye, V, H):
    CW = 12800
    grid = (V + CW - 1) // CW
    return pl.pallas_call(
        functools.partial(_pack_body, CW, H),
        grid=(grid,),
        in_specs=[
            pl.BlockSpec((H, CW), lambda i: (0, i)),
            pl.BlockSpec((H, H), lambda i: (0, 0)),
        ],
        out_specs=pl.BlockSpec((CW // 2, 2 * H), lambda i: (i, 0)),
        out_shape=jax.ShapeDtypeStruct((V // 2, 2 * H), jnp.float32),
        scratch_shapes=[pltpu.VMEM((CW, H), jnp.float32)],
        compiler_params=pltpu.CompilerParams(
            dimension_semantics=("arbitrary",),
        ),
    )(embT, eye)


def _gru_step_body(H, S, e_ref, w_ref, b_ref, eye_ref,
                   out_ref, hn_ref, h_scr):
    s = pl.program_id(1)

    @pl.when(s == 0)
    def _init():
        h_scr[...] = jnp.zeros_like(h_scr)

    h = h_scr[...]
    x_pair = e_ref[0]
    x_B = jax.lax.dot_general(
        eye_ref[...], x_pair, (((1,), (1,)), ((), ())),
        preferred_element_type=jnp.float32,
    )
    x_T = jnp.concatenate([x_B[0:H], x_B[H:2 * H]], axis=1)
    a = jnp.concatenate([x_T, h], axis=0)
    g = jnp.dot(w_ref[...], a, preferred_element_type=jnp.float32) + b_ref[...]
    r = jax.nn.sigmoid(g[0:H])
    z = jax.nn.sigmoid(g[H:2 * H])
    n = jnp.tanh(g[2 * H:3 * H] + r * g[3 * H:4 * H])
    h_new = (1.0 - z) * n + z * h
    h_scr[...] = h_new
    out_ref[0] = h_new

    @pl.when(s == S - 1)
    def _fin():
        hn_ref[0] = h_new


def _gru_tc(e_sbp, w2, b2, eye, B, S, H, BB):
    NB = B // BB
    body = functools.partial(_gru_step_body, H, S)
    return pl.pallas_call(
        body,
        grid=(NB, S),
        in_specs=[
            pl.BlockSpec((1, BB // 2, 2 * H), lambda b, s: (s, b, 0)),
            pl.BlockSpec((4 * H, 2 * H), lambda b, s: (0, 0)),
            pl.BlockSpec((4 * H, 1), lambda b, s: (0, 0)),
            pl.BlockSpec((2 * H, 2 * H), lambda b, s: (0, 0)),
        ],
        out_specs=[
            pl.BlockSpec((1, H, BB), lambda b, s: (s, 0, b)),
            pl.BlockSpec((1, H, BB), lambda b, s: (0, 0, b)),
        ],
        out_shape=[
            jax.ShapeDtypeStruct((S, H, B), jnp.float32),
            jax.ShapeDtypeStruct((1, H, B), jnp.float32),
        ],
        scratch_shapes=[pltpu.VMEM((H, BB), jnp.float32)],
        compiler_params=pltpu.CompilerParams(
            dimension_semantics=("parallel", "arbitrary"),
        ),
    )(e_sbp, w2, b2, eye)


def kernel(x, emb, W_ih, W_hh, b_ih, b_hh):
    B, S = x.shape
    V, H = emb.shape

    eye = jnp.eye(H, dtype=jnp.float32)
    eye2 = jnp.eye(2 * H, dtype=jnp.float32)
    table = _tc_pack(emb.T, eye, V, H).reshape(V, H)

    BB = min(512, B)
    NB = B // BB
    BH = BB // 2
    idx = (
        x.T.reshape(S, NB, 2, BH)
        .transpose(0, 1, 3, 2)
        .reshape(-1)
        .astype(jnp.int32)
    )

    e_flat = _sc_gather(table, idx, B * S, H)
    e_sbp = e_flat.reshape(S, B // 2, 2 * H)

    zeros = jnp.zeros((H, H), jnp.float32)
    left = jnp.concatenate([W_ih[:2 * H], W_ih[2 * H:], zeros], axis=0)
    right = jnp.concatenate([W_hh[:2 * H], zeros, W_hh[2 * H:]], axis=0)
    w2 = jnp.concatenate([left, right], axis=1)
    b2 = jnp.concatenate(
        [b_ih[:2 * H] + b_hh[:2 * H], b_ih[2 * H:], b_hh[2 * H:]]
    )[:, None]

    out_shb, hn_hb = _gru_tc(e_sbp, w2, b2, eye2, B, S, H, BB)
    out = jnp.transpose(out_shb, (2, 0, 1))
    h_n = jnp.transpose(hn_hb, (0, 2, 1))
    return out, h_n


# speedup vs baseline: 2.1294x; 1.2381x over previous
import functools

import jax
import jax.numpy as jnp
from jax import lax
from jax.experimental import pallas as pl
from jax.experimental.pallas import tpu as pltpu
from jax.experimental.pallas import tpu_sc as plsc


def _sc_gather(packed, idx, n_rows, row_w):
    NW = 32
    rows_per_w = n_rows // NW
    STREAM = 128
    K = 10
    GROUP = K * STREAM
    NGRP = rows_per_w // GROUP

    mesh = plsc.VectorSubcoreMesh(core_axis_name="c", subcore_axis_name="s")

    @functools.partial(
        pl.kernel,
        mesh=mesh,
        out_type=jax.ShapeDtypeStruct((n_rows, row_w), jnp.float32),
        scratch_types=[
            pltpu.VMEM((rows_per_w,), jnp.int32),
            pltpu.VMEM((GROUP, row_w), jnp.float32),
            pltpu.SemaphoreType.DMA,
        ],
        compiler_params=pltpu.CompilerParams(use_tc_tiling_on_sc=False),
    )
    def gather_k(tab_hbm, idx_hbm, out_hbm, idx_v, rows_v, sem):
        cid = lax.axis_index("c")
        sid = lax.axis_index("s")
        wid = sid * 2 + cid
        base = wid * rows_per_w
        pltpu.sync_copy(idx_hbm.at[pl.ds(base, rows_per_w)], idx_v)

        def grp(g, carry):
            goff = g * GROUP
            handles = []
            for j in range(K):
                handles.append(
                    pltpu.async_copy(
                        tab_hbm.at[idx_v.at[pl.ds(goff + j * STREAM, STREAM)]],
                        rows_v.at[pl.ds(j * STREAM, STREAM)],
                        sem,
                    )
                )
            for h in handles:
                h.wait()
            pltpu.sync_copy(rows_v, out_hbm.at[pl.ds(base + goff, GROUP)])
            return carry

        lax.fori_loop(0, NGRP, grp, 0)

    return gather_k(packed, idx)


def _pack_body(cw, h, et_ref, eye_ref, o_ref, t_scr):
    chunk = et_ref[...]
    t_scr[...] = jax.lax.dot_general(
        chunk, eye_ref[...], (((0,), (0,)), ((), ())),
        preferred_element_type=jnp.float32,
    )
    even = t_scr[pl.Slice(0, cw // 2, 2), :]
    odd = t_scr[pl.Slice(1, cw // 2, 2), :]
    o_ref[...] = jnp.concatenate([even, odd], axis=1)


def _tc_pack(embT, eye, V, H):
    CW = 12800
    grid = (V + CW - 1) // CW
    return pl.pallas_call(
        functools.partial(_pack_body, CW, H),
        grid=(grid,),
        in_specs=[
            pl.BlockSpec((H, CW), lambda i: (0, i)),
            pl.BlockSpec((H, H), lambda i: (0, 0)),
        ],
        out_specs=pl.BlockSpec((CW // 2, 2 * H), lambda i: (i, 0)),
        out_shape=jax.ShapeDtypeStruct((V // 2, 2 * H), jnp.float32),
        scratch_shapes=[pltpu.VMEM((CW, H), jnp.float32)],
        compiler_params=pltpu.CompilerParams(
            dimension_semantics=("arbitrary",),
        ),
    )(embT, eye)


def _gru_step_body(H, S, e_ref, w_ref, b_ref, eye_ref,
                   out_ref, hn_ref, h_scr):
    s = pl.program_id(1)

    @pl.when(s == 0)
    def _init():
        h_scr[...] = jnp.zeros_like(h_scr)

    h = h_scr[...]
    x_pair = e_ref[0]
    x_B = jax.lax.dot_general(
        eye_ref[...], x_pair, (((1,), (1,)), ((), ())),
        preferred_element_type=jnp.float32,
    )
    x_T = jnp.concatenate([x_B[0:H], x_B[H:2 * H]], axis=1)
    a = jnp.concatenate([x_T, h], axis=0)
    g = jnp.dot(w_ref[...], a, preferred_element_type=jnp.float32) + b_ref[...]
    r = jax.nn.sigmoid(g[0:H])
    z = jax.nn.sigmoid(g[H:2 * H])
    n = jnp.tanh(g[2 * H:3 * H] + r * g[3 * H:4 * H])
    h_new = (1.0 - z) * n + z * h
    h_scr[...] = h_new
    out_ref[0] = h_new

    @pl.when(s == S - 1)
    def _fin():
        hn_ref[0] = h_new


def _gru_tc(e_sbp, w2, b2, eye, B, S, H, BB):
    NB = B // BB
    body = functools.partial(_gru_step_body, H, S)
    return pl.pallas_call(
        body,
        grid=(NB, S),
        in_specs=[
            pl.BlockSpec((1, BB // 2, 2 * H), lambda b, s: (s, b, 0)),
            pl.BlockSpec((4 * H, 2 * H), lambda b, s: (0, 0)),
            pl.BlockSpec((4 * H, 1), lambda b, s: (0, 0)),
            pl.BlockSpec((2 * H, 2 * H), lambda b, s: (0, 0)),
        ],
        out_specs=[
            pl.BlockSpec((1, H, BB), lambda b, s: (s, 0, b)),
            pl.BlockSpec((1, H, BB), lambda b, s: (0, 0, b)),
        ],
        out_shape=[
            jax.ShapeDtypeStruct((S, H, B), jnp.float32),
            jax.ShapeDtypeStruct((1, H, B), jnp.float32),
        ],
        scratch_shapes=[pltpu.VMEM((H, BB), jnp.float32)],
        compiler_params=pltpu.CompilerParams(
            dimension_semantics=("parallel", "arbitrary"),
        ),
    )(e_sbp, w2, b2, eye)


def kernel(x, emb, W_ih, W_hh, b_ih, b_hh):
    B, S = x.shape
    V, H = emb.shape

    eye = jnp.eye(H, dtype=jnp.float32)
    eye2 = jnp.eye(2 * H, dtype=jnp.float32)
    table = _tc_pack(emb.T, eye, V, H).reshape(V, H)

    BB = min(1024, B)
    NB = B // BB
    BH = BB // 2
    idx = (
        x.T.reshape(S, NB, 2, BH)
        .transpose(0, 1, 3, 2)
        .reshape(-1)
        .astype(jnp.int32)
    )

    e_flat = _sc_gather(table, idx, B * S, H)
    e_sbp = e_flat.reshape(S, B // 2, 2 * H)

    zeros = jnp.zeros((H, H), jnp.float32)
    left = jnp.concatenate([W_ih[:2 * H], W_ih[2 * H:], zeros], axis=0)
    right = jnp.concatenate([W_hh[:2 * H], zeros, W_hh[2 * H:]], axis=0)
    w2 = jnp.concatenate([left, right], axis=1)
    b2 = jnp.concatenate(
        [b_ih[:2 * H] + b_hh[:2 * H], b_ih[2 * H:], b_hh[2 * H:]]
    )[:, None]

    out_shb, hn_hb = _gru_tc(e_sbp, w2, b2, eye2, B, S, H, BB)
    out = jnp.transpose(out_shb, (2, 0, 1))
    h_n = jnp.transpose(hn_hb, (0, 2, 1))
    return out, h_n


# speedup vs baseline: 2.4129x; 1.1332x over previous
import functools

import jax
import jax.numpy as jnp
from jax import lax
from jax.experimental import pallas as pl
from jax.experimental.pallas import tpu as pltpu
from jax.experimental.pallas import tpu_sc as plsc


def _sc_gather(packed, idx, n_rows, row_w):
    NW = 32
    rows_per_w = n_rows // NW
    STREAM = 128
    K = 10
    GROUP = K * STREAM
    NGRP = rows_per_w // GROUP

    mesh = plsc.VectorSubcoreMesh(core_axis_name="c", subcore_axis_name="s")

    @functools.partial(
        pl.kernel,
        mesh=mesh,
        out_type=jax.ShapeDtypeStruct((n_rows, row_w), jnp.float32),
        scratch_types=[
            pltpu.VMEM((rows_per_w,), jnp.int32),
            pltpu.VMEM((GROUP, row_w), jnp.float32),
            pltpu.SemaphoreType.DMA,
        ],
        compiler_params=pltpu.CompilerParams(use_tc_tiling_on_sc=False),
    )
    def gather_k(tab_hbm, idx_hbm, out_hbm, idx_v, rows_v, sem):
        cid = lax.axis_index("c")
        sid = lax.axis_index("s")
        wid = sid * 2 + cid
        base = wid * rows_per_w
        pltpu.sync_copy(idx_hbm.at[pl.ds(base, rows_per_w)], idx_v)

        def grp(g, carry):
            goff = g * GROUP
            handles = []
            for j in range(K):
                handles.append(
                    pltpu.async_copy(
                        tab_hbm.at[idx_v.at[pl.ds(goff + j * STREAM, STREAM)]],
                        rows_v.at[pl.ds(j * STREAM, STREAM)],
                        sem,
                    )
                )
            for h in handles:
                h.wait()
            pltpu.sync_copy(rows_v, out_hbm.at[pl.ds(base + goff, GROUP)])
            return carry

        lax.fori_loop(0, NGRP, grp, 0)

    return gather_k(packed, idx)


def _pack_body(cw, h, et_ref, eye_ref, o_ref, t_scr):
    chunk = et_ref[...]
    t_scr[...] = jax.lax.dot_general(
        chunk, eye_ref[...], (((0,), (0,)), ((), ())),
        preferred_element_type=jnp.float32,
    )
    even = t_scr[pl.Slice(0, cw // 2, 2), :]
    odd = t_scr[pl.Slice(1, cw // 2, 2), :]
    o_ref[...] = jnp.concatenate([even, odd], axis=1)


def _tc_pack(embT, eye, V, H):
    CW = 12800
    grid = (V + CW - 1) // CW
    return pl.pallas_call(
        functools.partial(_pack_body, CW, H),
        grid=(grid,),
        in_specs=[
            pl.BlockSpec((H, CW), lambda i: (0, i)),
            pl.BlockSpec((H, H), lambda i: (0, 0)),
        ],
        out_specs=pl.BlockSpec((CW // 2, 2 * H), lambda i: (i, 0)),
        out_shape=jax.ShapeDtypeStruct((V // 2, 2 * H), jnp.float32),
        scratch_shapes=[pltpu.VMEM((CW, H), jnp.float32)],
        compiler_params=pltpu.CompilerParams(
            dimension_semantics=("arbitrary",),
        ),
    )(embT, eye)


def _gru_step_body(H, S, e_ref, w_ref, b_ref, eye_ref,
                   out_ref, hn_ref, h_scr):
    s = pl.program_id(1)

    @pl.when(s == 0)
    def _init():
        h_scr[...] = jnp.zeros_like(h_scr)

    h = h_scr[...]
    x_pair = e_ref[0]
    x_B = jax.lax.dot_general(
        eye_ref[...], x_pair, (((1,), (1,)), ((), ())),
        preferred_element_type=jnp.float32,
    )
    x_T = jnp.concatenate([x_B[0:H], x_B[H:2 * H]], axis=1)
    a = jnp.concatenate([x_T, h], axis=0)
    g = jnp.dot(w_ref[...], a, preferred_element_type=jnp.float32) + b_ref[...]
    r = jax.nn.sigmoid(g[0:H])
    z = jax.nn.sigmoid(g[H:2 * H])
    n = jnp.tanh(g[2 * H:3 * H] + r * g[3 * H:4 * H])
    h_new = (1.0 - z) * n + z * h
    h_scr[...] = h_new
    out_ref[0] = h_new

    @pl.when(s == S - 1)
    def _fin():
        hn_ref[0] = h_new


def _gru_tc(e_sbp, w2, b2, eye, B, S, H, BB):
    NB = B // BB
    body = functools.partial(_gru_step_body, H, S)
    return pl.pallas_call(
        body,
        grid=(NB, S),
        in_specs=[
            pl.BlockSpec((1, BB // 2, 2 * H), lambda b, s: (s, b, 0)),
            pl.BlockSpec((4 * H, 2 * H), lambda b, s: (0, 0)),
            pl.BlockSpec((4 * H, 1), lambda b, s: (0, 0)),
            pl.BlockSpec((2 * H, 2 * H), lambda b, s: (0, 0)),
        ],
        out_specs=[
            pl.BlockSpec((1, H, BB), lambda b, s: (s, 0, b)),
            pl.BlockSpec((1, H, BB), lambda b, s: (0, 0, b)),
        ],
        out_shape=[
            jax.ShapeDtypeStruct((S, H, B), jnp.float32),
            jax.ShapeDtypeStruct((1, H, B), jnp.float32),
        ],
        scratch_shapes=[pltpu.VMEM((H, BB), jnp.float32)],
        compiler_params=pltpu.CompilerParams(
            dimension_semantics=("parallel", "arbitrary"),
        ),
    )(e_sbp, w2, b2, eye)


def kernel(x, emb, W_ih, W_hh, b_ih, b_hh):
    B, S = x.shape
    V, H = emb.shape

    eye = jnp.eye(H, dtype=jnp.float32)
    eye2 = jnp.eye(2 * H, dtype=jnp.float32)
    table = _tc_pack(emb.T, eye, V, H).reshape(V, H)

    BB = min(2048, B)
    NB = B // BB
    BH = BB // 2
    idx = (
        x.T.reshape(S, NB, 2, BH)
        .transpose(0, 1, 3, 2)
        .reshape(-1)
        .astype(jnp.int32)
    )

    e_flat = _sc_gather(table, idx, B * S, H)
    e_sbp = e_flat.reshape(S, B // 2, 2 * H)

    zeros = jnp.zeros((H, H), jnp.float32)
    left = jnp.concatenate([W_ih[:2 * H], W_ih[2 * H:], zeros], axis=0)
    right = jnp.concatenate([W_hh[:2 * H], zeros, W_hh[2 * H:]], axis=0)
    w2 = jnp.concatenate([left, right], axis=1)
    b2 = jnp.concatenate(
        [b_ih[:2 * H] + b_hh[:2 * H], b_ih[2 * H:], b_hh[2 * H:]]
    )[:, None]

    out_shb, hn_hb = _gru_tc(e_sbp, w2, b2, eye2, B, S, H, BB)
    out = jnp.transpose(out_shb, (2, 0, 1))
    h_n = jnp.transpose(hn_hb, (0, 2, 1))
    return out, h_n


# speedup vs baseline: 2.5879x; 1.0725x over previous
import functools

import jax
import jax.numpy as jnp
from jax import lax
from jax.experimental import pallas as pl
from jax.experimental.pallas import tpu as pltpu
from jax.experimental.pallas import tpu_sc as plsc


def _sc_gather(packed, idx, n_rows, row_w):
    NW = 32
    rows_per_w = n_rows // NW
    STREAM = 128
    K = 10
    GROUP = K * STREAM
    NGRP = rows_per_w // GROUP

    mesh = plsc.VectorSubcoreMesh(core_axis_name="c", subcore_axis_name="s")

    @functools.partial(
        pl.kernel,
        mesh=mesh,
        out_type=jax.ShapeDtypeStruct((n_rows, row_w), jnp.float32),
        scratch_types=[
            pltpu.VMEM((rows_per_w,), jnp.int32),
            pltpu.VMEM((GROUP, row_w), jnp.float32),
            pltpu.SemaphoreType.DMA,
        ],
        compiler_params=pltpu.CompilerParams(use_tc_tiling_on_sc=False),
    )
    def gather_k(tab_hbm, idx_hbm, out_hbm, idx_v, rows_v, sem):
        cid = lax.axis_index("c")
        sid = lax.axis_index("s")
        wid = sid * 2 + cid
        base = wid * rows_per_w
        pltpu.sync_copy(idx_hbm.at[pl.ds(base, rows_per_w)], idx_v)

        def grp(g, carry):
            goff = g * GROUP
            handles = []
            for j in range(K):
                handles.append(
                    pltpu.async_copy(
                        tab_hbm.at[idx_v.at[pl.ds(goff + j * STREAM, STREAM)]],
                        rows_v.at[pl.ds(j * STREAM, STREAM)],
                        sem,
                    )
                )
            for h in handles:
                h.wait()
            pltpu.sync_copy(rows_v, out_hbm.at[pl.ds(base + goff, GROUP)])
            return carry

        lax.fori_loop(0, NGRP, grp, 0)

    return gather_k(packed, idx)


def _pack_body(cw, h, et_ref, eye_ref, o_ref, t_scr):
    chunk = et_ref[...]
    t_scr[...] = jax.lax.dot_general(
        chunk, eye_ref[...], (((0,), (0,)), ((), ())),
        preferred_element_type=jnp.float32,
    )
    even = t_scr[pl.Slice(0, cw // 2, 2), :]
    odd = t_scr[pl.Slice(1, cw // 2, 2), :]
    o_ref[...] = jnp.concatenate([even, odd], axis=1)


def _tc_pack(embT, eye, V, H):
    CW = 12800
    grid = (V + CW - 1) // CW
    return pl.pallas_call(
        functools.partial(_pack_body, CW, H),
        grid=(grid,),
        in_specs=[
            pl.BlockSpec((H, CW), lambda i: (0, i)),
            pl.BlockSpec((H, H), lambda i: (0, 0)),
        ],
        out_specs=pl.BlockSpec((CW // 2, 2 * H), lambda i: (i, 0)),
        out_shape=jax.ShapeDtypeStruct((V // 2, 2 * H), jnp.float32),
        scratch_shapes=[pltpu.VMEM((CW, H), jnp.float32)],
        compiler_params=pltpu.CompilerParams(
            dimension_semantics=("arbitrary",),
        ),
    )(embT, eye)


def _gru_step_body(H, S, e_ref, w_ref, b_ref, eye_ref,
                   out_ref, hn_ref, h_scr):
    s = pl.program_id(1)

    @pl.when(s == 0)
    def _init():
        h_scr[...] = jnp.zeros_like(h_scr)

    h = h_scr[...]
    x_pair = e_ref[0]
    x_B = jax.lax.dot_general(
        eye_ref[...], x_pair, (((1,), (1,)), ((), ())),
        preferred_element_type=jnp.float32,
    )
    x_T = jnp.concatenate([x_B[0:H], x_B[H:2 * H]], axis=1)
    a = jnp.concatenate([x_T, h], axis=0)
    g = jnp.dot(w_ref[...], a, preferred_element_type=jnp.float32) + b_ref[...]
    r = jax.nn.sigmoid(g[0:H])
    z = jax.nn.sigmoid(g[H:2 * H])
    n = jnp.tanh(g[2 * H:3 * H] + r * g[3 * H:4 * H])
    h_new = (1.0 - z) * n + z * h
    h_scr[...] = h_new
    out_ref[0] = h_new

    @pl.when(s == S - 1)
    def _fin():
        hn_ref[0] = h_new


def _gru_tc(e_sbp, w2, b2, eye, B, S, H, BB):
    NB = B // BB
    body = functools.partial(_gru_step_body, H, S)
    return pl.pallas_call(
        body,
        grid=(NB, S),
        in_specs=[
            pl.BlockSpec((1, BB // 2, 2 * H), lambda b, s: (s, b, 0)),
            pl.BlockSpec((4 * H, 2 * H), lambda b, s: (0, 0)),
            pl.BlockSpec((4 * H, 1), lambda b, s: (0, 0)),
            pl.BlockSpec((2 * H, 2 * H), lambda b, s: (0, 0)),
        ],
        out_specs=[
            pl.BlockSpec((1, H, BB), lambda b, s: (s, 0, b)),
            pl.BlockSpec((1, H, BB), lambda b, s: (0, 0, b)),
        ],
        out_shape=[
            jax.ShapeDtypeStruct((S, H, B), jnp.float32),
            jax.ShapeDtypeStruct((1, H, B), jnp.float32),
        ],
        scratch_shapes=[pltpu.VMEM((H, BB), jnp.float32)],
        compiler_params=pltpu.CompilerParams(
            dimension_semantics=("parallel", "arbitrary"),
        ),
    )(e_sbp, w2, b2, eye)


def kernel(x, emb, W_ih, W_hh, b_ih, b_hh):
    B, S = x.shape
    V, H = emb.shape

    eye = jnp.eye(H, dtype=jnp.float32)
    eye2 = jnp.eye(2 * H, dtype=jnp.float32)
    table = _tc_pack(emb.T, eye, V, H).reshape(V, H)

    BB = min(4096, B)
    NB = B // BB
    BH = BB // 2
    idx = (
        x.T.reshape(S, NB, 2, BH)
        .transpose(0, 1, 3, 2)
        .reshape(-1)
        .astype(jnp.int32)
    )

    e_flat = _sc_gather(table, idx, B * S, H)
    e_sbp = e_flat.reshape(S, B // 2, 2 * H)

    zeros = jnp.zeros((H, H), jnp.float32)
    left = jnp.concatenate([W_ih[:2 * H], W_ih[2 * H:], zeros], axis=0)
    right = jnp.concatenate([W_hh[:2 * H], zeros, W_hh[2 * H:]], axis=0)
    w2 = jnp.concatenate([left, right], axis=1)
    b2 = jnp.concatenate(
        [b_ih[:2 * H] + b_hh[:2 * H], b_ih[2 * H:], b_hh[2 * H:]]
    )[:, None]

    out_shb, hn_hb = _gru_tc(e_sbp, w2, b2, eye2, B, S, H, BB)
    out = jnp.transpose(out_shb, (2, 0, 1))
    h_n = jnp.transpose(hn_hb, (0, 2, 1))
    return out, h_n


# speedup vs baseline: 2.6000x; 1.0046x over previous
import functools

import jax
import jax.numpy as jnp
from jax import lax
from jax.experimental import pallas as pl
from jax.experimental.pallas import tpu as pltpu
from jax.experimental.pallas import tpu_sc as plsc


def _sc_gather(packed, idx, n_rows, row_w):
    NW = 32
    rows_per_w = n_rows // NW
    STREAM = 128
    K = 10
    GROUP = K * STREAM
    NGRP = rows_per_w // GROUP

    mesh = plsc.VectorSubcoreMesh(core_axis_name="c", subcore_axis_name="s")

    @functools.partial(
        pl.kernel,
        mesh=mesh,
        out_type=jax.ShapeDtypeStruct((n_rows, row_w), jnp.float32),
        scratch_types=[
            pltpu.VMEM((rows_per_w,), jnp.int32),
            pltpu.VMEM((GROUP, row_w), jnp.float32),
            pltpu.SemaphoreType.DMA,
        ],
        compiler_params=pltpu.CompilerParams(use_tc_tiling_on_sc=False),
    )
    def gather_k(tab_hbm, idx_hbm, out_hbm, idx_v, rows_v, sem):
        cid = lax.axis_index("c")
        sid = lax.axis_index("s")
        wid = sid * 2 + cid
        base = wid * rows_per_w
        pltpu.sync_copy(idx_hbm.at[pl.ds(base, rows_per_w)], idx_v)

        def grp(g, carry):
            goff = g * GROUP
            handles = []
            for j in range(K):
                handles.append(
                    pltpu.async_copy(
                        tab_hbm.at[idx_v.at[pl.ds(goff + j * STREAM, STREAM)]],
                        rows_v.at[pl.ds(j * STREAM, STREAM)],
                        sem,
                    )
                )
            for h in handles:
                h.wait()
            pltpu.sync_copy(rows_v, out_hbm.at[pl.ds(base + goff, GROUP)])
            return carry

        lax.fori_loop(0, NGRP, grp, 0)

    return gather_k(packed, idx)


def _pack_body(cw, h, et_ref, eye_ref, o_ref, t_scr):
    chunk = et_ref[...]
    t_scr[...] = jnp.swapaxes(chunk, 0, 1)
    even = t_scr[pl.Slice(0, cw // 2, 2), :]
    odd = t_scr[pl.Slice(1, cw // 2, 2), :]
    o_ref[...] = jnp.concatenate([even, odd], axis=1)


def _tc_pack(embT, eye, V, H):
    CW = 12800
    grid = (V + CW - 1) // CW
    return pl.pallas_call(
        functools.partial(_pack_body, CW, H),
        grid=(grid,),
        in_specs=[
            pl.BlockSpec((H, CW), lambda i: (0, i)),
            pl.BlockSpec((H, H), lambda i: (0, 0)),
        ],
        out_specs=pl.BlockSpec((CW // 2, 2 * H), lambda i: (i, 0)),
        out_shape=jax.ShapeDtypeStruct((V // 2, 2 * H), jnp.float32),
        scratch_shapes=[pltpu.VMEM((CW, H), jnp.float32)],
        compiler_params=pltpu.CompilerParams(
            dimension_semantics=("arbitrary",),
        ),
    )(embT, eye)


def _gru_step_body(H, S, e_ref, w_ref, b_ref, eye_ref,
                   out_ref, hn_ref, h_scr):
    s = pl.program_id(1)

    @pl.when(s == 0)
    def _init():
        h_scr[...] = jnp.zeros_like(h_scr)

    h = h_scr[...]
    x_pair = e_ref[0]
    x_B = jax.lax.dot_general(
        eye_ref[...], x_pair, (((1,), (1,)), ((), ())),
        preferred_element_type=jnp.float32,
    )
    x_T = jnp.concatenate([x_B[0:H], x_B[H:2 * H]], axis=1)
    a = jnp.concatenate([x_T, h], axis=0)
    g = jnp.dot(w_ref[...], a, preferred_element_type=jnp.float32) + b_ref[...]
    r = jax.nn.sigmoid(g[0:H])
    z = jax.nn.sigmoid(g[H:2 * H])
    n = jnp.tanh(g[2 * H:3 * H] + r * g[3 * H:4 * H])
    h_new = (1.0 - z) * n + z * h
    h_scr[...] = h_new
    out_ref[0] = h_new

    @pl.when(s == S - 1)
    def _fin():
        hn_ref[0] = h_new


def _gru_tc(e_sbp, w2, b2, eye, B, S, H, BB):
    NB = B // BB
    body = functools.partial(_gru_step_body, H, S)
    return pl.pallas_call(
        body,
        grid=(NB, S),
        in_specs=[
            pl.BlockSpec((1, BB // 2, 2 * H), lambda b, s: (s, b, 0)),
            pl.BlockSpec((4 * H, 2 * H), lambda b, s: (0, 0)),
            pl.BlockSpec((4 * H, 1), lambda b, s: (0, 0)),
            pl.BlockSpec((2 * H, 2 * H), lambda b, s: (0, 0)),
        ],
        out_specs=[
            pl.BlockSpec((1, H, BB), lambda b, s: (s, 0, b)),
            pl.BlockSpec((1, H, BB), lambda b, s: (0, 0, b)),
        ],
        out_shape=[
            jax.ShapeDtypeStruct((S, H, B), jnp.float32),
            jax.ShapeDtypeStruct((1, H, B), jnp.float32),
        ],
        scratch_shapes=[pltpu.VMEM((H, BB), jnp.float32)],
        compiler_params=pltpu.CompilerParams(
            dimension_semantics=("parallel", "arbitrary"),
        ),
    )(e_sbp, w2, b2, eye)


def kernel(x, emb, W_ih, W_hh, b_ih, b_hh):
    B, S = x.shape
    V, H = emb.shape

    eye = jnp.eye(H, dtype=jnp.float32)
    eye2 = jnp.eye(2 * H, dtype=jnp.float32)
    table = _tc_pack(emb.T, eye, V, H).reshape(V, H)

    BB = min(4096, B)
    NB = B // BB
    BH = BB // 2
    idx = (
        x.T.reshape(S, NB, 2, BH)
        .transpose(0, 1, 3, 2)
        .reshape(-1)
        .astype(jnp.int32)
    )

    e_flat = _sc_gather(table, idx, B * S, H)
    e_sbp = e_flat.reshape(S, B // 2, 2 * H)

    zeros = jnp.zeros((H, H), jnp.float32)
    left = jnp.concatenate([W_ih[:2 * H], W_ih[2 * H:], zeros], axis=0)
    right = jnp.concatenate([W_hh[:2 * H], zeros, W_hh[2 * H:]], axis=0)
    w2 = jnp.concatenate([left, right], axis=1)
    b2 = jnp.concatenate(
        [b_ih[:2 * H] + b_hh[:2 * H], b_ih[2 * H:], b_hh[2 * H:]]
    )[:, None]

    out_shb, hn_hb = _gru_tc(e_sbp, w2, b2, eye2, B, S, H, BB)
    out = jnp.transpose(out_shb, (2, 0, 1))
    h_n = jnp.transpose(hn_hb, (0, 2, 1))
    return out, h_n
